# Initial kernel scaffold; baseline (speedup 1.0000x reference)
import jax
import jax.numpy as jnp
from jax.experimental import pallas as pl


def kernel(x, edge_index, W1, b1, W2, b2):
    raise NotImplementedError("write your pallas kernel here")


# speedup vs baseline: 11.6563x; 11.6563x over previous
import functools

import jax
import jax.numpy as jnp
from jax import lax
from jax.experimental import pallas as pl
from jax.experimental.pallas import tpu as pltpu
from jax.experimental.pallas import tpu_sc as plsc

N = 10000
E = 320000
F = 128
PN = 10240
NC = 2
NS = 16
CH = 80

_MESH = plsc.VectorSubcoreMesh(
    core_axis_name="c", subcore_axis_name="s", num_cores=NC, num_subcores=NS
)


def _fill(ref, n, value):
    vec = jnp.full((16,), value, jnp.float32)

    def body(i, _):
        ref[pl.ds(i * 16, 16)] = vec
        return 0

    lax.fori_loop(0, n // 16, body, 0)


@functools.partial(
    pl.kernel,
    out_type=jax.ShapeDtypeStruct((PN,), jnp.float32),
    mesh=_MESH,
    scratch_types=[
        pltpu.VMEM((CH,), jnp.int32),
        pltpu.VMEM((CH,), jnp.float32),
        pltpu.VMEM((640,), jnp.float32),
        pltpu.VMEM_SHARED((PN,), jnp.float32),
    ],
)
def _k1(dst_hbm, hist_hbm, idx_v, ones_v, zbuf, acc):
    cid = lax.axis_index("c")
    sid = lax.axis_index("s")
    _fill(ones_v, CH, 1.0)
    _fill(zbuf, 640, 0.0)
    pltpu.sync_copy(zbuf, acc.at[pl.ds(sid * 640, 640)])
    plsc.subcore_barrier()

    per_w = E // NS

    def body(c, _):
        base = sid * per_w + c * CH
        pltpu.sync_copy(dst_hbm.at[pl.ds(base, CH)], idx_v)
        pltpu.sync_copy(ones_v, acc.at[idx_v], add=True)
        return 0

    lax.fori_loop(0, per_w // CH, body, 0)
    plsc.subcore_barrier()

    @pl.when(cid == 0)
    def _():
        pltpu.sync_copy(acc.at[pl.ds(sid * 640, 640)],
                        hist_hbm.at[pl.ds(sid * 640, 640)])


def _k2_body(hist_ref, x_ref, w1_ref, y_ref, dinv_ref):
    dinv = lax.rsqrt(hist_ref[...] + 1.0)
    xw = jnp.dot(x_ref[...], w1_ref[...], preferred_element_type=jnp.float32)
    y_ref[pl.ds(0, N), :] = xw * dinv[:N, None]
    y_ref[pl.ds(N, PN - N), :] = jnp.zeros((PN - N, F), jnp.float32)
    dinv_ref[...] = dinv


def _k2(hist, x, W1):
    return pl.pallas_call(
        _k2_body,
        out_shape=(
            jax.ShapeDtypeStruct((PN, F), jnp.float32),
            jax.ShapeDtypeStruct((PN,), jnp.float32),
        ),
    )(hist, x, W1)


@functools.partial(
    pl.kernel,
    out_type=jax.ShapeDtypeStruct((NC, PN, F), jnp.float32),
    mesh=_MESH,
    scratch_types=[
        pltpu.VMEM((CH,), jnp.int32),
        pltpu.VMEM((CH,), jnp.int32),
        pltpu.VMEM((CH, F), jnp.float32),
        pltpu.SemaphoreType.DMA,
        pltpu.VMEM_SHARED((PN, F), jnp.float32),
    ],
)
def _k3(src_hbm, dst_hbm, y_hbm, out_hbm, idxs, idxd, rows, sem, acc):
    cid = lax.axis_index("c")
    sid = lax.axis_index("s")
    w = cid * NS + sid

    def zrow(i, _):
        rows[i, pl.ds(0, 16)] = jnp.zeros((16,), jnp.float32)
        rows[i, pl.ds(16, 16)] = jnp.zeros((16,), jnp.float32)
        rows[i, pl.ds(32, 16)] = jnp.zeros((16,), jnp.float32)
        rows[i, pl.ds(48, 16)] = jnp.zeros((16,), jnp.float32)
        rows[i, pl.ds(64, 16)] = jnp.zeros((16,), jnp.float32)
        rows[i, pl.ds(80, 16)] = jnp.zeros((16,), jnp.float32)
        rows[i, pl.ds(96, 16)] = jnp.zeros((16,), jnp.float32)
        rows[i, pl.ds(112, 16)] = jnp.zeros((16,), jnp.float32)
        return 0

    lax.fori_loop(0, CH, zrow, 0)
    for q in range(8):
        pltpu.sync_copy(rows, acc.at[pl.ds(sid * 640 + q * CH, CH)])
    plsc.subcore_barrier()

    per_w = E // (NC * NS)

    def body(c, _):
        base = w * per_w + c * CH
        pltpu.sync_copy(src_hbm.at[pl.ds(base, CH)], idxs)
        pltpu.sync_copy(dst_hbm.at[pl.ds(base, CH)], idxd)
        pltpu.async_copy(y_hbm.at[idxs], rows, sem).wait()
        pltpu.sync_copy(rows, acc.at[idxd], add=True)
        return 0

    lax.fori_loop(0, per_w // CH, body, 0)
    plsc.subcore_barrier()
    pltpu.sync_copy(acc.at[pl.ds(sid * 640, 640)],
                    out_hbm.at[cid, pl.ds(sid * 640, 640)])


def _k4_body(p_ref, y_ref, dinv_ref, b1_ref, w2_ref, b2_ref, zs_ref, u_ref):
    dinv = dinv_ref[...]
    agg = p_ref[0] + p_ref[1] + y_ref[...]
    h = jax.nn.relu(agg * dinv[:, None] + b1_ref[...][None, :])
    z = jnp.dot(h, w2_ref[...], preferred_element_type=jnp.float32)[:, 0]
    zs = z * dinv
    zs_ref[...] = zs
    u_ref[...] = dinv * zs + b2_ref[0]


def _k4(partials, y, dinv, b1, W2, b2):
    return pl.pallas_call(
        _k4_body,
        out_shape=(
            jax.ShapeDtypeStruct((PN,), jnp.float32),
            jax.ShapeDtypeStruct((PN,), jnp.float32),
        ),
    )(partials, y, dinv, b1, W2, b2)


@functools.partial(
    pl.kernel,
    out_type=jax.ShapeDtypeStruct((PN,), jnp.float32),
    mesh=_MESH,
    scratch_types=[
        pltpu.VMEM((CH,), jnp.int32),
        pltpu.VMEM((CH,), jnp.int32),
        pltpu.VMEM((CH,), jnp.float32),
        pltpu.SemaphoreType.DMA,
        pltpu.VMEM((640,), jnp.float32),
        pltpu.VMEM((640,), jnp.float32),
        pltpu.VMEM((640,), jnp.float32),
        pltpu.VMEM_SHARED((PN,), jnp.float32),
    ],
)
def _k5(src_hbm, dst_hbm, zs_hbm, dinv_hbm, u_hbm, out_hbm,
        idxs, idxd, vals, sem, abuf, dbuf, ubuf, acc):
    cid = lax.axis_index("c")
    sid = lax.axis_index("s")
    _fill(abuf, 640, 0.0)
    pltpu.sync_copy(abuf, acc.at[pl.ds(sid * 640, 640)])
    plsc.subcore_barrier()

    per_w = E // NS

    def body(c, _):
        base = sid * per_w + c * CH
        pltpu.sync_copy(src_hbm.at[pl.ds(base, CH)], idxs)
        pltpu.sync_copy(dst_hbm.at[pl.ds(base, CH)], idxd)
        pltpu.async_copy(zs_hbm.at[idxs], vals, sem).wait()
        pltpu.sync_copy(vals, acc.at[idxd], add=True)
        return 0

    lax.fori_loop(0, per_w // CH, body, 0)
    plsc.subcore_barrier()

    @pl.when(cid == 0)
    def _():
        base = sid * 640
        pltpu.sync_copy(acc.at[pl.ds(base, 640)], abuf)
        pltpu.sync_copy(dinv_hbm.at[pl.ds(base, 640)], dbuf)
        pltpu.sync_copy(u_hbm.at[pl.ds(base, 640)], ubuf)

        def comb(i, _):
            sl = pl.ds(i * 16, 16)
            ubuf[sl] = abuf[sl] * dbuf[sl] + ubuf[sl]
            return 0

        lax.fori_loop(0, 40, comb, 0)
        pltpu.sync_copy(ubuf, out_hbm.at[pl.ds(base, 640)])


def kernel(x, edge_index, W1, b1, W2, b2):
    src = edge_index[0]
    dst = edge_index[1]
    hist = _k1(dst)
    y, dinv = _k2(hist, x, W1)
    partials = _k3(src, dst, y)
    zs, u = _k4(partials, y, dinv, b1, W2, b2)
    out = _k5(src, dst, zs, dinv, u)
    return out[:N]


# speedup vs baseline: 33.3391x; 2.8602x over previous
import functools

import jax
import jax.numpy as jnp
from jax import lax
from jax.experimental import pallas as pl
from jax.experimental.pallas import tpu as pltpu
from jax.experimental.pallas import tpu_sc as plsc

N = 10000
E = 320000
F = 128
PN = 10240
NC = 2
NS = 16
CH = 80
NCHUNK = 125
G = 5
NGROUP = NCHUNK // G

_MESH = plsc.VectorSubcoreMesh(
    core_axis_name="c", subcore_axis_name="s", num_cores=NC, num_subcores=NS
)


def _fill(ref, n, value):
    vec = jnp.full((16,), value, jnp.float32)

    def body(i, _):
        ref[pl.ds(i * 16, 16)] = vec
        return 0

    lax.fori_loop(0, n // 16, body, 0)


@functools.partial(
    pl.kernel,
    out_type=jax.ShapeDtypeStruct((PN,), jnp.float32),
    mesh=_MESH,
    scratch_types=[
        pltpu.VMEM((2 * NCHUNK, CH), jnp.int32),
        pltpu.VMEM((CH,), jnp.float32),
        pltpu.VMEM((640,), jnp.float32),
        pltpu.SemaphoreType.DMA,
        pltpu.VMEM_SHARED((PN,), jnp.float32),
    ],
)
def _k1(dst3_hbm, hist_hbm, idx_v, ones_v, zbuf, sem, acc):
    cid = lax.axis_index("c")
    sid = lax.axis_index("s")
    _fill(ones_v, CH, 1.0)
    _fill(zbuf, 640, 0.0)
    pltpu.sync_copy(zbuf, acc.at[pl.ds(sid * 640, 640)])
    pltpu.sync_copy(dst3_hbm.at[2 * sid], idx_v.at[pl.ds(0, NCHUNK)])
    pltpu.sync_copy(dst3_hbm.at[2 * sid + 1], idx_v.at[pl.ds(NCHUNK, NCHUNK)])
    plsc.subcore_barrier()

    def body(c, _):
        pltpu.async_copy(ones_v, acc.at[idx_v.at[c]], sem, add=True)
        return 0

    lax.fori_loop(0, 2 * NCHUNK, body, 0)

    def drain(c, _):
        pltpu.make_async_copy(ones_v, acc.at[idx_v.at[c]], sem).wait()
        return 0

    lax.fori_loop(0, 2 * NCHUNK, drain, 0)
    plsc.subcore_barrier()

    @pl.when(cid == 0)
    def _():
        pltpu.sync_copy(acc.at[pl.ds(sid * 640, 640)],
                        hist_hbm.at[pl.ds(sid * 640, 640)])


def _k2_body(hist_ref, x_ref, w1_ref, y_ref, dinv_ref):
    dinv = lax.rsqrt(hist_ref[...] + 1.0)
    xw = jnp.dot(x_ref[...], w1_ref[...], preferred_element_type=jnp.float32)
    y_ref[pl.ds(0, N), :] = xw * dinv[:N, None]
    y_ref[pl.ds(N, PN - N), :] = jnp.zeros((PN - N, F), jnp.float32)
    dinv_ref[...] = dinv


def _k2(hist, x, W1):
    return pl.pallas_call(
        _k2_body,
        out_shape=(
            jax.ShapeDtypeStruct((PN, F), jnp.float32),
            jax.ShapeDtypeStruct((PN,), jnp.float32),
        ),
    )(hist, x, W1)


@functools.partial(
    pl.kernel,
    out_type=jax.ShapeDtypeStruct((NC, PN, F), jnp.float32),
    mesh=_MESH,
    scratch_types=[
        pltpu.VMEM((NCHUNK, CH), jnp.int32),
        pltpu.VMEM((NCHUNK, CH), jnp.int32),
        pltpu.VMEM((CH, F), jnp.float32),
        pltpu.SemaphoreType.DMA,
        pltpu.SemaphoreType.DMA,
        pltpu.VMEM_SHARED((PN, F), jnp.float32),
    ],
)
def _k3(src3_hbm, dst3_hbm, y_hbm, out_hbm, idxs, idxd, rows, gsem, ssem, acc):
    cid = lax.axis_index("c")
    sid = lax.axis_index("s")
    w = cid * NS + sid

    def zrow(i, _):
        for k in range(F // 16):
            rows[i, pl.ds(16 * k, 16)] = jnp.zeros((16,), jnp.float32)
        return 0

    lax.fori_loop(0, CH, zrow, 0)
    for q in range(8):
        pltpu.sync_copy(rows, acc.at[pl.ds(sid * 640 + q * CH, CH)])
    pltpu.sync_copy(src3_hbm.at[w], idxs)
    pltpu.sync_copy(dst3_hbm.at[w], idxd)
    plsc.subcore_barrier()

    def body(c, _):
        pltpu.async_copy(y_hbm.at[idxs.at[c]], rows, gsem).wait()
        pltpu.async_copy(rows, acc.at[idxd.at[c]], ssem, add=True).wait()
        return 0

    lax.fori_loop(0, NCHUNK, body, 0)
    plsc.subcore_barrier()
    pltpu.sync_copy(acc.at[pl.ds(sid * 640, 640)],
                    out_hbm.at[cid, pl.ds(sid * 640, 640)])


def _k4_body(p_ref, y_ref, dinv_ref, b1_ref, w2_ref, b2_ref, zs_ref, u_ref):
    dinv = dinv_ref[...]
    agg = p_ref[0] + p_ref[1] + y_ref[...]
    h = jax.nn.relu(agg * dinv[:, None] + b1_ref[...][None, :])
    z = jnp.dot(h, w2_ref[...], preferred_element_type=jnp.float32)[:, 0]
    zs = z * dinv
    zs_ref[...] = zs
    u_ref[...] = dinv * zs + b2_ref[0]


def _k4(partials, y, dinv, b1, W2, b2):
    return pl.pallas_call(
        _k4_body,
        out_shape=(
            jax.ShapeDtypeStruct((PN,), jnp.float32),
            jax.ShapeDtypeStruct((PN,), jnp.float32),
        ),
    )(partials, y, dinv, b1, W2, b2)


@functools.partial(
    pl.kernel,
    out_type=jax.ShapeDtypeStruct((NC, PN), jnp.float32),
    mesh=_MESH,
    scratch_types=[
        pltpu.VMEM((NCHUNK, CH), jnp.int32),
        pltpu.VMEM((NCHUNK, CH), jnp.int32),
        pltpu.VMEM((G * CH,), jnp.float32),
        pltpu.SemaphoreType.DMA,
        pltpu.SemaphoreType.DMA,
        pltpu.VMEM((640,), jnp.float32),
        pltpu.VMEM_SHARED((PN,), jnp.float32),
    ],
)
def _k5(src3_hbm, dst3_hbm, zs_hbm, out_hbm,
        idxs, idxd, vals, gsem, ssem, zbuf, acc):
    cid = lax.axis_index("c")
    sid = lax.axis_index("s")
    w = cid * NS + sid
    _fill(zbuf, 640, 0.0)
    pltpu.sync_copy(zbuf, acc.at[pl.ds(sid * 640, 640)])
    pltpu.sync_copy(src3_hbm.at[w], idxs)
    pltpu.sync_copy(dst3_hbm.at[w], idxd)
    plsc.subcore_barrier()

    def group(g, _):
        gds = []
        for j in range(G):
            gds.append(pltpu.async_copy(
                zs_hbm.at[idxs.at[g * G + j]],
                vals.at[pl.ds(j * CH, CH)], gsem))
        for d in gds:
            d.wait()
        sds = []
        for j in range(G):
            sds.append(pltpu.async_copy(
                vals.at[pl.ds(j * CH, CH)],
                acc.at[idxd.at[g * G + j]], ssem, add=True))
        for d in sds:
            d.wait()
        return 0

    lax.fori_loop(0, NGROUP, group, 0)
    plsc.subcore_barrier()
    pltpu.sync_copy(acc.at[pl.ds(sid * 640, 640)],
                    out_hbm.at[cid, pl.ds(sid * 640, 640)])


def _k6_body(p_ref, dinv_ref, u_ref, o_ref):
    o_ref[...] = dinv_ref[...] * (p_ref[0] + p_ref[1]) + u_ref[...]


def _k6(partials2, dinv, u):
    return pl.pallas_call(
        _k6_body,
        out_shape=jax.ShapeDtypeStruct((PN,), jnp.float32),
    )(partials2, dinv, u)


def kernel(x, edge_index, W1, b1, W2, b2):
    src3 = edge_index[0].reshape(NC * NS, NCHUNK, CH)
    dst3 = edge_index[1].reshape(NC * NS, NCHUNK, CH)
    hist = _k1(dst3)
    y, dinv = _k2(hist, x, W1)
    partials = _k3(src3, dst3, y)
    zs, u = _k4(partials, y, dinv, b1, W2, b2)
    partials2 = _k5(src3, dst3, zs)
    out = _k6(partials2, dinv, u)
    return out[:N]


# speedup vs baseline: 39.3218x; 1.1794x over previous
import functools

import jax
import jax.numpy as jnp
from jax import lax
from jax.experimental import pallas as pl
from jax.experimental.pallas import tpu as pltpu
from jax.experimental.pallas import tpu_sc as plsc

N = 10000
E = 320000
F = 128
PN = 10240
NC = 2
NS = 16
CH = 80
NCHUNK = 125
G = 5
NGROUP = NCHUNK // G

_MESH = plsc.VectorSubcoreMesh(
    core_axis_name="c", subcore_axis_name="s", num_cores=NC, num_subcores=NS
)


def _fill(ref, n, value):
    vec = jnp.full((16,), value, jnp.float32)

    def body(i, _):
        ref[pl.ds(i * 16, 16)] = vec
        return 0

    lax.fori_loop(0, n // 16, body, 0)


@functools.partial(
    pl.kernel,
    out_type=jax.ShapeDtypeStruct((PN,), jnp.float32),
    mesh=_MESH,
    scratch_types=[
        pltpu.VMEM((2 * NCHUNK, CH), jnp.int32),
        pltpu.VMEM((CH,), jnp.float32),
        pltpu.VMEM((640,), jnp.float32),
        pltpu.SemaphoreType.DMA,
        pltpu.VMEM_SHARED((PN,), jnp.float32),
    ],
)
def _k1(dst3_hbm, hist_hbm, idx_v, ones_v, zbuf, sem, acc):
    cid = lax.axis_index("c")
    sid = lax.axis_index("s")
    _fill(ones_v, CH, 1.0)
    _fill(zbuf, 640, 0.0)
    pltpu.sync_copy(zbuf, acc.at[pl.ds(sid * 640, 640)])
    pltpu.sync_copy(dst3_hbm.at[2 * sid], idx_v.at[pl.ds(0, NCHUNK)])
    pltpu.sync_copy(dst3_hbm.at[2 * sid + 1], idx_v.at[pl.ds(NCHUNK, NCHUNK)])
    plsc.subcore_barrier()

    def body(c, _):
        pltpu.async_copy(ones_v, acc.at[idx_v.at[c]], sem, add=True)
        return 0

    lax.fori_loop(0, 2 * NCHUNK, body, 0)

    def drain(c, _):
        pltpu.make_async_copy(ones_v, acc.at[idx_v.at[c]], sem).wait()
        return 0

    lax.fori_loop(0, 2 * NCHUNK, drain, 0)
    plsc.subcore_barrier()

    @pl.when(cid == 0)
    def _():
        pltpu.sync_copy(acc.at[pl.ds(sid * 640, 640)],
                        hist_hbm.at[pl.ds(sid * 640, 640)])


def _k2_body(hist_ref, x_ref, w1_ref, y_ref, dinv_ref):
    dinv = lax.rsqrt(hist_ref[...] + 1.0)
    xw = jnp.dot(x_ref[...], w1_ref[...], preferred_element_type=jnp.float32)
    y_ref[pl.ds(0, N), :] = xw * dinv[:N, None]
    y_ref[pl.ds(N, PN - N), :] = jnp.zeros((PN - N, F), jnp.float32)
    dinv_ref[...] = dinv


def _k2(hist, x, W1):
    return pl.pallas_call(
        _k2_body,
        out_shape=(
            jax.ShapeDtypeStruct((PN, F), jnp.float32),
            jax.ShapeDtypeStruct((PN,), jnp.float32),
        ),
    )(hist, x, W1)


@functools.partial(
    pl.kernel,
    out_type=jax.ShapeDtypeStruct((NC, PN, F), jnp.float32),
    mesh=_MESH,
    scratch_types=[
        pltpu.VMEM((NCHUNK * CH,), jnp.int32),
        pltpu.VMEM((NCHUNK, CH), jnp.int32),
        pltpu.VMEM((CH, F), jnp.float32),
        pltpu.VMEM((CH, F), jnp.float32),
        pltpu.SemaphoreType.DMA,
        pltpu.SemaphoreType.DMA,
        pltpu.VMEM_SHARED((PN, F), jnp.float32),
    ],
)
def _k3(src1_hbm, dst3_hbm, y_hbm, out_hbm, idxs, idxd, bufa, bufb,
        gsa, gsb, acc):
    cid = lax.axis_index("c")
    sid = lax.axis_index("s")
    w = cid * NS + sid

    def zrow(i, _):
        for k in range(F // 16):
            bufa[i, pl.ds(16 * k, 16)] = jnp.zeros((16,), jnp.float32)
        return 0

    lax.fori_loop(0, CH, zrow, 0)
    for q in range(8):
        pltpu.sync_copy(bufa, acc.at[pl.ds(sid * 640 + q * CH, CH)])
    pltpu.sync_copy(src1_hbm.at[pl.ds(w * NCHUNK * CH, NCHUNK * CH)], idxs)
    pltpu.sync_copy(dst3_hbm.at[w], idxd)
    plsc.subcore_barrier()

    def gidx(c):
        return idxs.at[pl.ds(c * CH, CH)]

    pltpu.async_copy(y_hbm.at[gidx(0)], bufa, gsa)

    def pair(g, _):
        c = 2 * g
        pltpu.make_async_copy(y_hbm.at[gidx(c)], bufa, gsa).wait()
        pltpu.async_copy(y_hbm.at[gidx(c + 1)], bufb, gsb)
        pltpu.sync_copy(bufa, acc.at[idxd.at[c]], add=True)
        pltpu.make_async_copy(y_hbm.at[gidx(c + 1)], bufb, gsb).wait()
        pltpu.async_copy(y_hbm.at[gidx(c + 2)], bufa, gsa)
        pltpu.sync_copy(bufb, acc.at[idxd.at[c + 1]], add=True)
        return 0

    lax.fori_loop(0, (NCHUNK - 1) // 2, pair, 0)
    pltpu.make_async_copy(y_hbm.at[gidx(NCHUNK - 1)], bufa, gsa).wait()
    pltpu.sync_copy(bufa, acc.at[idxd.at[NCHUNK - 1]], add=True)
    plsc.subcore_barrier()
    pltpu.sync_copy(acc.at[pl.ds(sid * 640, 640)],
                    out_hbm.at[cid, pl.ds(sid * 640, 640)])


def _k4_body(p_ref, y_ref, dinv_ref, b1_ref, w2_ref, b2_ref, zs_ref, u_ref):
    dinv = dinv_ref[...]
    agg = p_ref[0] + p_ref[1] + y_ref[...]
    h = jax.nn.relu(agg * dinv[:, None] + b1_ref[...][None, :])
    z = jnp.dot(h, w2_ref[...], preferred_element_type=jnp.float32)[:, 0]
    zs = z * dinv
    zs_ref[...] = zs
    u_ref[...] = dinv * zs + b2_ref[0]


def _k4(partials, y, dinv, b1, W2, b2):
    return pl.pallas_call(
        _k4_body,
        out_shape=(
            jax.ShapeDtypeStruct((PN,), jnp.float32),
            jax.ShapeDtypeStruct((PN,), jnp.float32),
        ),
    )(partials, y, dinv, b1, W2, b2)


@functools.partial(
    pl.kernel,
    out_type=jax.ShapeDtypeStruct((NC, PN), jnp.float32),
    mesh=_MESH,
    scratch_types=[
        pltpu.VMEM((NCHUNK, CH), jnp.int32),
        pltpu.VMEM((NCHUNK, CH), jnp.int32),
        pltpu.VMEM((G * CH,), jnp.float32),
        pltpu.SemaphoreType.DMA,
        pltpu.SemaphoreType.DMA,
        pltpu.VMEM((640,), jnp.float32),
        pltpu.VMEM_SHARED((PN,), jnp.float32),
    ],
)
def _k5(src3_hbm, dst3_hbm, zs_hbm, out_hbm,
        idxs, idxd, vals, gsem, ssem, zbuf, acc):
    cid = lax.axis_index("c")
    sid = lax.axis_index("s")
    w = cid * NS + sid
    _fill(zbuf, 640, 0.0)
    pltpu.sync_copy(zbuf, acc.at[pl.ds(sid * 640, 640)])
    pltpu.sync_copy(src3_hbm.at[w], idxs)
    pltpu.sync_copy(dst3_hbm.at[w], idxd)
    plsc.subcore_barrier()

    def group(g, _):
        gds = []
        for j in range(G):
            gds.append(pltpu.async_copy(
                zs_hbm.at[idxs.at[g * G + j]],
                vals.at[pl.ds(j * CH, CH)], gsem))
        for d in gds:
            d.wait()
        sds = []
        for j in range(G):
            sds.append(pltpu.async_copy(
                vals.at[pl.ds(j * CH, CH)],
                acc.at[idxd.at[g * G + j]], ssem, add=True))
        for d in sds:
            d.wait()
        return 0

    lax.fori_loop(0, NGROUP, group, 0)
    plsc.subcore_barrier()
    pltpu.sync_copy(acc.at[pl.ds(sid * 640, 640)],
                    out_hbm.at[cid, pl.ds(sid * 640, 640)])


def _k6_body(p_ref, dinv_ref, u_ref, o_ref):
    o_ref[...] = dinv_ref[...] * (p_ref[0] + p_ref[1]) + u_ref[...]


def _k6(partials2, dinv, u):
    return pl.pallas_call(
        _k6_body,
        out_shape=jax.ShapeDtypeStruct((PN,), jnp.float32),
    )(partials2, dinv, u)


def kernel(x, edge_index, W1, b1, W2, b2):
    src3 = edge_index[0].reshape(NC * NS, NCHUNK, CH)
    dst3 = edge_index[1].reshape(NC * NS, NCHUNK, CH)
    hist = _k1(dst3)
    y, dinv = _k2(hist, x, W1)
    partials = _k3(edge_index[0], dst3, y)
    zs, u = _k4(partials, y, dinv, b1, W2, b2)
    partials2 = _k5(src3, dst3, zs)
    out = _k6(partials2, dinv, u)
    return out[:N]


# speedup vs baseline: 44.8200x; 1.1398x over previous
import functools

import jax
import jax.numpy as jnp
from jax import lax
from jax.experimental import pallas as pl
from jax.experimental.pallas import tpu as pltpu
from jax.experimental.pallas import tpu_sc as plsc

N = 10000
E = 320000
F = 128
PN = 10240
NC = 2
NS = 16
CH = 80
NCHUNK = 125
G = 5
NGROUP = NCHUNK // G
G5 = 25

_MESH = plsc.VectorSubcoreMesh(
    core_axis_name="c", subcore_axis_name="s", num_cores=NC, num_subcores=NS
)


def _fill(ref, n, value):
    vec = jnp.full((16,), value, jnp.float32)

    def body(i, _):
        ref[pl.ds(i * 16, 16)] = vec
        return 0

    lax.fori_loop(0, n // 16, body, 0)


@functools.partial(
    pl.kernel,
    out_type=jax.ShapeDtypeStruct((PN,), jnp.float32),
    mesh=_MESH,
    scratch_types=[
        pltpu.VMEM((2 * NCHUNK, CH), jnp.int32),
        pltpu.VMEM((CH,), jnp.float32),
        pltpu.VMEM((640,), jnp.float32),
        pltpu.SemaphoreType.DMA,
        pltpu.VMEM_SHARED((PN,), jnp.float32),
    ],
)
def _k1(dst3_hbm, hist_hbm, idx_v, ones_v, zbuf, sem, acc):
    cid = lax.axis_index("c")
    sid = lax.axis_index("s")
    _fill(ones_v, CH, 1.0)
    _fill(zbuf, 640, 0.0)
    pltpu.sync_copy(zbuf, acc.at[pl.ds(sid * 640, 640)])
    pltpu.sync_copy(dst3_hbm.at[2 * sid], idx_v.at[pl.ds(0, NCHUNK)])
    pltpu.sync_copy(dst3_hbm.at[2 * sid + 1], idx_v.at[pl.ds(NCHUNK, NCHUNK)])
    plsc.subcore_barrier()

    def body(c, _):
        pltpu.async_copy(ones_v, acc.at[idx_v.at[c]], sem, add=True)
        return 0

    lax.fori_loop(0, 2 * NCHUNK, body, 0)

    def drain(c, _):
        pltpu.make_async_copy(ones_v, acc.at[idx_v.at[c]], sem).wait()
        return 0

    lax.fori_loop(0, 2 * NCHUNK, drain, 0)
    plsc.subcore_barrier()

    @pl.when(cid == 0)
    def _():
        pltpu.sync_copy(acc.at[pl.ds(sid * 640, 640)],
                        hist_hbm.at[pl.ds(sid * 640, 640)])


def _k2_body(hist_ref, x_ref, w1_ref, y_ref, dinv_ref):
    dinv = lax.rsqrt(hist_ref[...] + 1.0)
    xw = jnp.dot(x_ref[...], w1_ref[...], preferred_element_type=jnp.float32)
    y_ref[pl.ds(0, N), :] = xw * dinv[:N, None]
    y_ref[pl.ds(N, PN - N), :] = jnp.zeros((PN - N, F), jnp.float32)
    dinv_ref[...] = dinv


def _k2(hist, x, W1):
    return pl.pallas_call(
        _k2_body,
        out_shape=(
            jax.ShapeDtypeStruct((PN, F), jnp.float32),
            jax.ShapeDtypeStruct((PN,), jnp.float32),
        ),
    )(hist, x, W1)


@functools.partial(
    pl.kernel,
    out_type=jax.ShapeDtypeStruct((NC, PN, F), jnp.float32),
    mesh=_MESH,
    scratch_types=[
        pltpu.VMEM((NCHUNK * CH,), jnp.int32),
        pltpu.VMEM((NCHUNK, CH), jnp.int32),
        pltpu.VMEM((CH, F), jnp.float32),
        pltpu.VMEM((CH, F), jnp.float32),
        pltpu.SemaphoreType.DMA,
        pltpu.SemaphoreType.DMA,
        pltpu.VMEM_SHARED((PN, F), jnp.float32),
    ],
)
def _k3(src1_hbm, dst3_hbm, y_hbm, out_hbm, idxs, idxd, bufa, bufb,
        gsa, gsb, acc):
    cid = lax.axis_index("c")
    sid = lax.axis_index("s")
    w = cid * NS + sid

    def zrow(i, _):
        for k in range(F // 16):
            bufa[i, pl.ds(16 * k, 16)] = jnp.zeros((16,), jnp.float32)
        return 0

    lax.fori_loop(0, CH, zrow, 0)
    for q in range(8):
        pltpu.sync_copy(bufa, acc.at[pl.ds(sid * 640 + q * CH, CH)])
    pltpu.sync_copy(src1_hbm.at[pl.ds(w * NCHUNK * CH, NCHUNK * CH)], idxs)
    pltpu.sync_copy(dst3_hbm.at[w], idxd)
    plsc.subcore_barrier()

    def gidx(c):
        return idxs.at[pl.ds(c * CH, CH)]

    pltpu.async_copy(y_hbm.at[gidx(0)], bufa, gsa)

    def pair(g, _):
        c = 2 * g
        pltpu.make_async_copy(y_hbm.at[gidx(c)], bufa, gsa).wait()
        pltpu.async_copy(y_hbm.at[gidx(c + 1)], bufb, gsb)
        pltpu.sync_copy(bufa, acc.at[idxd.at[c]], add=True)
        pltpu.make_async_copy(y_hbm.at[gidx(c + 1)], bufb, gsb).wait()
        pltpu.async_copy(y_hbm.at[gidx(c + 2)], bufa, gsa)
        pltpu.sync_copy(bufb, acc.at[idxd.at[c + 1]], add=True)
        return 0

    lax.fori_loop(0, (NCHUNK - 1) // 2, pair, 0)
    pltpu.make_async_copy(y_hbm.at[gidx(NCHUNK - 1)], bufa, gsa).wait()
    pltpu.sync_copy(bufa, acc.at[idxd.at[NCHUNK - 1]], add=True)
    plsc.subcore_barrier()
    pltpu.sync_copy(acc.at[pl.ds(sid * 640, 640)],
                    out_hbm.at[cid, pl.ds(sid * 640, 640)])


def _k4_body(p_ref, y_ref, dinv_ref, b1_ref, w2_ref, b2_ref, zs_ref, u_ref):
    dinv = dinv_ref[...]
    agg = p_ref[0] + p_ref[1] + y_ref[...]
    h = jax.nn.relu(agg * dinv[:, None] + b1_ref[...][None, :])
    z = jnp.dot(h, w2_ref[...], preferred_element_type=jnp.float32)[:, 0]
    zs = z * dinv
    zs_ref[...] = zs
    u_ref[...] = dinv * zs + b2_ref[0]


def _k4(partials, y, dinv, b1, W2, b2):
    return pl.pallas_call(
        _k4_body,
        out_shape=(
            jax.ShapeDtypeStruct((PN,), jnp.float32),
            jax.ShapeDtypeStruct((PN,), jnp.float32),
        ),
    )(partials, y, dinv, b1, W2, b2)


@functools.partial(
    pl.kernel,
    out_type=jax.ShapeDtypeStruct((NC, PN), jnp.float32),
    mesh=_MESH,
    scratch_types=[
        pltpu.VMEM((NCHUNK, CH), jnp.int32),
        pltpu.VMEM((NCHUNK, CH), jnp.int32),
        pltpu.VMEM((G5 * CH,), jnp.float32),
        pltpu.SemaphoreType.DMA,
        pltpu.SemaphoreType.DMA,
        pltpu.VMEM((640,), jnp.float32),
        pltpu.VMEM_SHARED((PN,), jnp.float32),
        pltpu.VMEM_SHARED((PN,), jnp.float32),
    ],
)
def _k5(src3_hbm, dst3_hbm, zs_hbm, out_hbm,
        idxs, idxd, vals, gsem, ssem, zbuf, acc, ztab):
    cid = lax.axis_index("c")
    sid = lax.axis_index("s")
    w = cid * NS + sid
    _fill(zbuf, 640, 0.0)
    pltpu.sync_copy(zbuf, acc.at[pl.ds(sid * 640, 640)])
    pltpu.sync_copy(zs_hbm.at[pl.ds(sid * 640, 640)],
                    ztab.at[pl.ds(sid * 640, 640)])
    pltpu.sync_copy(src3_hbm.at[w], idxs)
    pltpu.sync_copy(dst3_hbm.at[w], idxd)
    plsc.subcore_barrier()

    def group(g, _):
        gds = []
        for j in range(G5):
            gds.append(pltpu.async_copy(
                ztab.at[idxs.at[g * G5 + j]],
                vals.at[pl.ds(j * CH, CH)], gsem))
        for d in gds:
            d.wait()
        sds = []
        for j in range(G5):
            sds.append(pltpu.async_copy(
                vals.at[pl.ds(j * CH, CH)],
                acc.at[idxd.at[g * G5 + j]], ssem, add=True))
        for d in sds:
            d.wait()
        return 0

    lax.fori_loop(0, NCHUNK // G5, group, 0)
    plsc.subcore_barrier()
    pltpu.sync_copy(acc.at[pl.ds(sid * 640, 640)],
                    out_hbm.at[cid, pl.ds(sid * 640, 640)])


def _k6_body(p_ref, dinv_ref, u_ref, o_ref):
    o_ref[...] = dinv_ref[...] * (p_ref[0] + p_ref[1]) + u_ref[...]


def _k6(partials2, dinv, u):
    return pl.pallas_call(
        _k6_body,
        out_shape=jax.ShapeDtypeStruct((PN,), jnp.float32),
    )(partials2, dinv, u)


def kernel(x, edge_index, W1, b1, W2, b2):
    src3 = edge_index[0].reshape(NC * NS, NCHUNK, CH)
    dst3 = edge_index[1].reshape(NC * NS, NCHUNK, CH)
    hist = _k1(dst3)
    y, dinv = _k2(hist, x, W1)
    partials = _k3(edge_index[0], dst3, y)
    zs, u = _k4(partials, y, dinv, b1, W2, b2)
    partials2 = _k5(src3, dst3, zs)
    out = _k6(partials2, dinv, u)
    return out[:N]
